# Initial kernel scaffold; baseline (speedup 1.0000x reference)
import jax
import jax.numpy as jnp
from jax.experimental import pallas as pl


def kernel(x, edge_index, W1, b1, W2, b2):
    raise NotImplementedError("write your pallas kernel here")


# speedup vs baseline: 2.5587x; 2.5587x over previous
import functools

import jax
import jax.numpy as jnp
from jax import lax
from jax.experimental import pallas as pl
from jax.experimental.pallas import tpu as pltpu
from jax.experimental.pallas import tpu_sc as plsc

N = 10000
NP = 10240
E = 320000
D = 128
NC, NS, L = 2, 16, 16
NW = NC * NS
EPW = E // NW
K = 80
NCH = EPW // K
BLK = 128
G = NP // BLK
RPS = NP // NS

_HI = lax.Precision.HIGHEST


def _prep_body(x_ref, w_ref, xn_ref, sup_ref):
    xb = x_ref[...]
    nrm2 = jnp.sum(xb * xb, axis=1, keepdims=True)
    scale = jnp.where(nrm2 > 0, lax.rsqrt(nrm2), 1.0)
    xn_ref[...] = xb * scale
    sup_ref[...] = jnp.dot(xb, w_ref[...], precision=_HI,
                           preferred_element_type=jnp.float32)


def _prep(xp, W):
    return pl.pallas_call(
        _prep_body,
        grid=(G,),
        in_specs=[pl.BlockSpec((BLK, D), lambda i: (i, 0)),
                  pl.BlockSpec((D, D), lambda i: (0, 0))],
        out_specs=[pl.BlockSpec((BLK, D), lambda i: (i, 0)),
                   pl.BlockSpec((BLK, D), lambda i: (i, 0))],
        out_shape=[jax.ShapeDtypeStruct((NP, D), jnp.float32),
                   jax.ShapeDtypeStruct((NP, D), jnp.float32)],
    )(xp, W)


def _stats_body(rs_ref, cnt_ref, invd_ref, wd_ref):
    rs = jnp.sum(rs_ref[...], axis=0, keepdims=True)
    deg = jnp.sum(cnt_ref[...], axis=0, keepdims=True)
    invd_ref[0] = jnp.where(rs == 0, 1.0, 1.0 / rs)
    wd_ref[0] = jnp.exp(1.0 / (deg + 1.0))


def _stats(rs, cnt):
    return pl.pallas_call(
        _stats_body,
        grid=(G,),
        in_specs=[pl.BlockSpec((NW, BLK), lambda i: (0, i)),
                  pl.BlockSpec((NW, BLK), lambda i: (0, i))],
        out_specs=[pl.BlockSpec((1, 1, BLK), lambda i: (i, 0, 0)),
                   pl.BlockSpec((1, 1, BLK), lambda i: (i, 0, 0))],
        out_shape=[jax.ShapeDtypeStruct((G, 1, BLK), jnp.float32),
                   jax.ShapeDtypeStruct((G, 1, BLK), jnp.float32)],
    )(rs, cnt)


def _col(wd_row):
    eye = jnp.eye(BLK, dtype=jnp.float32)
    return lax.dot_general(eye, wd_row, (((1,), (1,)), ((), ())),
                           precision=_HI)


def _mid_body(a0_ref, a1_ref, sup_ref, wd_ref, b_ref, w2_ref,
              hn_ref, sup2_ref):
    wd_col = _col(wd_ref[0])
    h = a0_ref[...] + a1_ref[...] + wd_col * sup_ref[...] + b_ref[...]
    h = jnp.maximum(h, 0.0)
    nrm2 = jnp.sum(h * h, axis=1, keepdims=True)
    scale = jnp.where(nrm2 > 0, lax.rsqrt(nrm2), 1.0)
    hn_ref[...] = h * scale
    sup2_ref[...] = jnp.dot(h, w2_ref[...], precision=_HI,
                            preferred_element_type=jnp.float32)


def _mid(a0, a1, sup, wd, brow, W2):
    return pl.pallas_call(
        _mid_body,
        grid=(G,),
        in_specs=[pl.BlockSpec((BLK, D), lambda i: (i, 0)),
                  pl.BlockSpec((BLK, D), lambda i: (i, 0)),
                  pl.BlockSpec((BLK, D), lambda i: (i, 0)),
                  pl.BlockSpec((1, 1, BLK), lambda i: (i, 0, 0)),
                  pl.BlockSpec((1, D), lambda i: (0, 0)),
                  pl.BlockSpec((D, D), lambda i: (0, 0))],
        out_specs=[pl.BlockSpec((BLK, D), lambda i: (i, 0)),
                   pl.BlockSpec((BLK, D), lambda i: (i, 0))],
        out_shape=[jax.ShapeDtypeStruct((NP, D), jnp.float32),
                   jax.ShapeDtypeStruct((NP, D), jnp.float32)],
    )(a0, a1, sup, wd, brow, W2)


def _fin_body(a0_ref, a1_ref, sup_ref, wd_ref, b_ref, out_ref):
    wd_col = _col(wd_ref[0])
    out_ref[...] = a0_ref[...] + a1_ref[...] + wd_col * sup_ref[...] + b_ref[...]


def _fin(a0, a1, sup, wd, brow):
    return pl.pallas_call(
        _fin_body,
        grid=(G,),
        in_specs=[pl.BlockSpec((BLK, D), lambda i: (i, 0)),
                  pl.BlockSpec((BLK, D), lambda i: (i, 0)),
                  pl.BlockSpec((BLK, D), lambda i: (i, 0)),
                  pl.BlockSpec((1, 1, BLK), lambda i: (i, 0, 0)),
                  pl.BlockSpec((1, D), lambda i: (0, 0))],
        out_specs=pl.BlockSpec((BLK, D), lambda i: (i, 0)),
        out_shape=jax.ShapeDtypeStruct((NP, D), jnp.float32),
    )(a0, a1, sup, wd, brow)


_MESH = plsc.VectorSubcoreMesh(core_axis_name="c", subcore_axis_name="s")


@functools.partial(
    pl.kernel,
    out_type=(
        jax.ShapeDtypeStruct((NW, NP), jnp.float32),
        jax.ShapeDtypeStruct((NW, NP), jnp.float32),
        jax.ShapeDtypeStruct((E,), jnp.float32),
    ),
    mesh=_MESH,
    compiler_params=pltpu.CompilerParams(needs_layout_passes=False),
    scratch_types=[
        pltpu.VMEM((K,), jnp.int32),
        pltpu.VMEM((K,), jnp.int32),
        pltpu.VMEM((K, D), jnp.float32),
        pltpu.VMEM((K, D), jnp.float32),
        pltpu.VMEM((K,), jnp.float32),
        pltpu.VMEM((NP,), jnp.float32),
        pltpu.VMEM((NP,), jnp.float32),
        pltpu.SemaphoreType.DMA,
        pltpu.SemaphoreType.DMA,
    ],
)
def _pass_a(xn_hbm, row_hbm, col_hbm, rs_hbm, cnt_hbm, sim_hbm,
            row_v, col_v, xr_v, xc_v, sim_v, rs_v, cnt_v, sem1, sem2):
    cid = lax.axis_index("c")
    sid = lax.axis_index("s")
    wid = sid * NC + cid
    base = wid * EPW
    zero16 = jnp.zeros((L,), jnp.float32)

    @pl.loop(0, NP // L)
    def _zero(i):
        rs_v[pl.ds(i * L, L)] = zero16
        cnt_v[pl.ds(i * L, L)] = zero16

    @pl.loop(0, NCH)
    def _chunk(ci):
        off = pl.multiple_of(base + ci * K, K)
        pltpu.sync_copy(row_hbm.at[pl.ds(off, K)], row_v)
        pltpu.sync_copy(col_hbm.at[pl.ds(off, K)], col_v)
        cp1 = pltpu.async_copy(xn_hbm.at[row_v], xr_v, sem1)
        cp2 = pltpu.async_copy(xn_hbm.at[col_v], xc_v, sem2)
        cp1.wait()
        cp2.wait()
        for g in range(K // L):
            evec = lax.iota(jnp.int32, L) + g * L

            def dstep(d, acc):
                dsplat = jnp.full((L,), d, jnp.int32)
                a = plsc.load_gather(xr_v, [evec, dsplat])
                b = plsc.load_gather(xc_v, [evec, dsplat])
                return acc + a * b

            acc = lax.fori_loop(0, D, dstep, zero16, unroll=4)
            sim16 = jnp.where(acc < 0.1, 0.0, acc)
            sim_v[pl.ds(g * L, L)] = sim16
            r16 = row_v[pl.ds(g * L, L)]
            plsc.addupdate_scatter(rs_v, [r16], sim16)
            plsc.addupdate_scatter(cnt_v, [r16],
                                   (sim16 > 0).astype(jnp.float32))
        pltpu.sync_copy(sim_v, sim_hbm.at[pl.ds(off, K)])

    pltpu.sync_copy(rs_v, rs_hbm.at[wid])
    pltpu.sync_copy(cnt_v, cnt_hbm.at[wid])


@functools.partial(
    pl.kernel,
    out_type=jax.ShapeDtypeStruct((NC, NP, D), jnp.float32),
    mesh=_MESH,
    compiler_params=pltpu.CompilerParams(needs_layout_passes=False),
    scratch_types=[
        pltpu.VMEM((K,), jnp.int32),
        pltpu.VMEM((K,), jnp.int32),
        pltpu.VMEM((K,), jnp.float32),
        pltpu.VMEM((K,), jnp.float32),
        pltpu.VMEM((K, D), jnp.float32),
        pltpu.VMEM((NP,), jnp.float32),
        pltpu.VMEM_SHARED((NP, D), jnp.float32),
        pltpu.SemaphoreType.DMA,
    ],
)
def _pass_b(sup_hbm, row_hbm, col_hbm, sim_hbm, invd_hbm, agg_hbm,
            row_v, col_v, sim_v, we_v, rows_v, invd_v, agg_sh, sem):
    cid = lax.axis_index("c")
    sid = lax.axis_index("s")
    wid = sid * NC + cid
    base = wid * EPW
    zero16 = jnp.zeros((L,), jnp.float32)
    pltpu.sync_copy(invd_hbm, invd_v)

    @pl.loop(0, K)
    def _zrows(i):
        for j in range(D // L):
            rows_v[i, pl.ds(j * L, L)] = zero16

    @pl.loop(0, RPS // K)
    def _zsh(j):
        pltpu.sync_copy(rows_v, agg_sh.at[pl.ds(sid * RPS + j * K, K)])

    plsc.subcore_barrier()

    @pl.loop(0, NCH)
    def _chunk(ci):
        off = pl.multiple_of(base + ci * K, K)
        pltpu.sync_copy(row_hbm.at[pl.ds(off, K)], row_v)
        pltpu.sync_copy(col_hbm.at[pl.ds(off, K)], col_v)
        pltpu.sync_copy(sim_hbm.at[pl.ds(off, K)], sim_v)
        cp = pltpu.async_copy(sup_hbm.at[col_v], rows_v, sem)
        for g in range(K // L):
            s16 = sim_v[pl.ds(g * L, L)]
            r16 = row_v[pl.ds(g * L, L)]
            iv16 = plsc.load_gather(invd_v, [r16])
            we = jnp.where(s16 > 0, jnp.exp(s16 * iv16), 0.0)
            we_v[pl.ds(g * L, L)] = we
        cp.wait()

        @pl.loop(0, K)
        def _scale(e):
            w = plsc.load_gather(we_v, [jnp.full((L,), e, jnp.int32)])
            for j in range(D // L):
                rows_v[e, pl.ds(j * L, L)] = rows_v[e, pl.ds(j * L, L)] * w

        pltpu.sync_copy(rows_v, agg_sh.at[row_v], add=True)

    plsc.subcore_barrier()

    @pl.loop(0, RPS // K)
    def _out(j):
        r0 = sid * RPS + j * K
        pltpu.sync_copy(agg_sh.at[pl.ds(r0, K)],
                        agg_hbm.at[cid, pl.ds(r0, K)])


def kernel(x, edge_index, W1, b1, W2, b2):
    row = edge_index[0]
    col = edge_index[1]
    xp = jnp.pad(x, ((0, NP - N), (0, 0)))
    b1r = b1.reshape(1, D)
    b2r = b2.reshape(1, D)

    xn, sup1 = _prep(xp, W1)
    rs1, cnt1, sim1 = _pass_a(xn, row, col)
    invd1, wd1 = _stats(rs1, cnt1)
    agg1 = _pass_b(sup1, row, col, sim1, invd1.reshape(NP))
    hn, sup2 = _mid(agg1[0], agg1[1], sup1, wd1, b1r, W2)
    rs2, cnt2, sim2 = _pass_a(hn, row, col)
    invd2, wd2 = _stats(rs2, cnt2)
    agg2 = _pass_b(sup2, row, col, sim2, invd2.reshape(NP))
    out = _fin(agg2[0], agg2[1], sup2, wd2, b2r)
    return out[:N]


# speedup vs baseline: 3.8572x; 1.5075x over previous
import functools

import jax
import jax.numpy as jnp
from jax import lax
from jax.experimental import pallas as pl
from jax.experimental.pallas import tpu as pltpu
from jax.experimental.pallas import tpu_sc as plsc

N = 10000
NP = 10240
E = 320000
D = 128
NC, NS, L = 2, 16, 16
NW = NC * NS
EPW = E // NW
K = 80
NCH = EPW // K
BLK = 128
G = NP // BLK
RPS = NP // NS

_HI = lax.Precision.HIGHEST


def _prep_body(x_ref, w_ref, xn_ref, sup_ref):
    xb = x_ref[...]
    nrm2 = jnp.sum(xb * xb, axis=1, keepdims=True)
    scale = jnp.where(nrm2 > 0, lax.rsqrt(nrm2), 1.0)
    xn_ref[...] = xb * scale
    sup_ref[...] = jnp.dot(xb, w_ref[...], precision=_HI,
                           preferred_element_type=jnp.float32)


def _prep(xp, W):
    return pl.pallas_call(
        _prep_body,
        grid=(G,),
        in_specs=[pl.BlockSpec((BLK, D), lambda i: (i, 0)),
                  pl.BlockSpec((D, D), lambda i: (0, 0))],
        out_specs=[pl.BlockSpec((BLK, D), lambda i: (i, 0)),
                   pl.BlockSpec((BLK, D), lambda i: (i, 0))],
        out_shape=[jax.ShapeDtypeStruct((NP, D), jnp.float32),
                   jax.ShapeDtypeStruct((NP, D), jnp.float32)],
    )(xp, W)


def _stats_body(rs_ref, cnt_ref, invd_ref, wd_ref):
    rs = jnp.sum(rs_ref[...], axis=0, keepdims=True)
    deg = jnp.sum(cnt_ref[...], axis=0, keepdims=True)
    invd_ref[0] = jnp.where(rs == 0, 1.0, 1.0 / rs)
    wd_ref[0] = jnp.exp(1.0 / (deg + 1.0))


def _stats(rs, cnt):
    return pl.pallas_call(
        _stats_body,
        grid=(G,),
        in_specs=[pl.BlockSpec((NW, BLK), lambda i: (0, i)),
                  pl.BlockSpec((NW, BLK), lambda i: (0, i))],
        out_specs=[pl.BlockSpec((1, 1, BLK), lambda i: (i, 0, 0)),
                   pl.BlockSpec((1, 1, BLK), lambda i: (i, 0, 0))],
        out_shape=[jax.ShapeDtypeStruct((G, 1, BLK), jnp.float32),
                   jax.ShapeDtypeStruct((G, 1, BLK), jnp.float32)],
    )(rs, cnt)


def _col(wd_row):
    eye = jnp.eye(BLK, dtype=jnp.float32)
    return lax.dot_general(eye, wd_row, (((1,), (1,)), ((), ())),
                           precision=_HI)


def _mid_body(a0_ref, a1_ref, sup_ref, wd_ref, b_ref, w2_ref,
              hn_ref, sup2_ref):
    wd_col = _col(wd_ref[0])
    h = a0_ref[...] + a1_ref[...] + wd_col * sup_ref[...] + b_ref[...]
    h = jnp.maximum(h, 0.0)
    nrm2 = jnp.sum(h * h, axis=1, keepdims=True)
    scale = jnp.where(nrm2 > 0, lax.rsqrt(nrm2), 1.0)
    hn_ref[...] = h * scale
    sup2_ref[...] = jnp.dot(h, w2_ref[...], precision=_HI,
                            preferred_element_type=jnp.float32)


def _mid(a0, a1, sup, wd, brow, W2):
    return pl.pallas_call(
        _mid_body,
        grid=(G,),
        in_specs=[pl.BlockSpec((BLK, D), lambda i: (i, 0)),
                  pl.BlockSpec((BLK, D), lambda i: (i, 0)),
                  pl.BlockSpec((BLK, D), lambda i: (i, 0)),
                  pl.BlockSpec((1, 1, BLK), lambda i: (i, 0, 0)),
                  pl.BlockSpec((1, D), lambda i: (0, 0)),
                  pl.BlockSpec((D, D), lambda i: (0, 0))],
        out_specs=[pl.BlockSpec((BLK, D), lambda i: (i, 0)),
                   pl.BlockSpec((BLK, D), lambda i: (i, 0))],
        out_shape=[jax.ShapeDtypeStruct((NP, D), jnp.float32),
                   jax.ShapeDtypeStruct((NP, D), jnp.float32)],
    )(a0, a1, sup, wd, brow, W2)


def _fin_body(a0_ref, a1_ref, sup_ref, wd_ref, b_ref, out_ref):
    wd_col = _col(wd_ref[0])
    out_ref[...] = a0_ref[...] + a1_ref[...] + wd_col * sup_ref[...] + b_ref[...]


def _fin(a0, a1, sup, wd, brow):
    return pl.pallas_call(
        _fin_body,
        grid=(G,),
        in_specs=[pl.BlockSpec((BLK, D), lambda i: (i, 0)),
                  pl.BlockSpec((BLK, D), lambda i: (i, 0)),
                  pl.BlockSpec((BLK, D), lambda i: (i, 0)),
                  pl.BlockSpec((1, 1, BLK), lambda i: (i, 0, 0)),
                  pl.BlockSpec((1, D), lambda i: (0, 0))],
        out_specs=pl.BlockSpec((BLK, D), lambda i: (i, 0)),
        out_shape=jax.ShapeDtypeStruct((NP, D), jnp.float32),
    )(a0, a1, sup, wd, brow)


_MESH = plsc.VectorSubcoreMesh(core_axis_name="c", subcore_axis_name="s")


def _dot_chunk(xr_v, xc_v, sim_v, row_v, rs_v, cnt_v, kbase):
    for g in range(K // L):
        evec = lax.iota(jnp.int32, L) + g * L

        def dstep(i, accs):
            a0, a1, a2, a3 = accs
            d = i * 4
            outs = []
            for k, acc in ((0, a0), (1, a1), (2, a2), (3, a3)):
                dsplat = jnp.full((L,), d + k, jnp.int32)
                a = plsc.load_gather(xr_v, [evec, dsplat])
                b = plsc.load_gather(xc_v, [evec, dsplat])
                outs.append(acc + a * b)
            return tuple(outs)

        z = jnp.zeros((L,), jnp.float32)
        a0, a1, a2, a3 = lax.fori_loop(0, D // 4, dstep, (z, z, z, z))
        acc = (a0 + a1) + (a2 + a3)
        sim16 = jnp.where(acc < 0.1, 0.0, acc)
        sim_v[pl.ds(kbase + g * L, L)] = sim16
        r16 = row_v[pl.ds(kbase + g * L, L)]
        plsc.addupdate_scatter(rs_v, [r16], sim16)
        plsc.addupdate_scatter(cnt_v, [r16],
                               (sim16 > 0).astype(jnp.float32))


@functools.partial(
    pl.kernel,
    out_type=(
        jax.ShapeDtypeStruct((NW, NP), jnp.float32),
        jax.ShapeDtypeStruct((NW, NP), jnp.float32),
        jax.ShapeDtypeStruct((E,), jnp.float32),
    ),
    mesh=_MESH,
    compiler_params=pltpu.CompilerParams(needs_layout_passes=False),
    scratch_types=[
        pltpu.VMEM((EPW,), jnp.int32),
        pltpu.VMEM((EPW,), jnp.int32),
        pltpu.VMEM((EPW,), jnp.float32),
        pltpu.VMEM((K, D), jnp.float32),
        pltpu.VMEM((K, D), jnp.float32),
        pltpu.VMEM((K, D), jnp.float32),
        pltpu.VMEM((K, D), jnp.float32),
        pltpu.VMEM((NP,), jnp.float32),
        pltpu.VMEM((NP,), jnp.float32),
        pltpu.SemaphoreType.DMA,
        pltpu.SemaphoreType.DMA,
        pltpu.SemaphoreType.DMA,
        pltpu.SemaphoreType.DMA,
    ],
)
def _pass_a(xn_hbm, row_hbm, col_hbm, rs_hbm, cnt_hbm, sim_hbm,
            row_v, col_v, sim_v, xr0, xc0, xr1, xc1,
            rs_v, cnt_v, semr0, semc0, semr1, semc1):
    cid = lax.axis_index("c")
    sid = lax.axis_index("s")
    wid = sid * NC + cid
    base = wid * EPW
    zero16 = jnp.zeros((L,), jnp.float32)
    xr = (xr0, xr1)
    xc = (xc0, xc1)
    semr = (semr0, semr1)
    semc = (semc0, semc1)

    pltpu.sync_copy(row_hbm.at[pl.ds(base, EPW)], row_v)
    pltpu.sync_copy(col_hbm.at[pl.ds(base, EPW)], col_v)

    @pl.loop(0, NP // L)
    def _zero(i):
        rs_v[pl.ds(i * L, L)] = zero16
        cnt_v[pl.ds(i * L, L)] = zero16

    def _issue(ci, b):
        off = pl.multiple_of(ci * K, K)
        pltpu.async_copy(xn_hbm.at[row_v.at[pl.ds(off, K)]], xr[b], semr[b])
        pltpu.async_copy(xn_hbm.at[col_v.at[pl.ds(off, K)]], xc[b], semc[b])

    def _wait(ci, b):
        off = pl.multiple_of(ci * K, K)
        pltpu.make_async_copy(xn_hbm.at[row_v.at[pl.ds(off, K)]],
                              xr[b], semr[b]).wait()
        pltpu.make_async_copy(xn_hbm.at[col_v.at[pl.ds(off, K)]],
                              xc[b], semc[b]).wait()

    _issue(0, 0)

    @pl.loop(0, (NCH - 1) // 2)
    def _chunk(j):
        for b in (0, 1):
            ci = j * 2 + b
            _wait(ci, b)
            _issue(ci + 1, 1 - b)
            _dot_chunk(xr[b], xc[b], sim_v, row_v, rs_v, cnt_v, ci * K)

    ci_last = NCH - 1
    _wait(ci_last, ci_last % 2)
    _dot_chunk(xr[ci_last % 2], xc[ci_last % 2], sim_v, row_v, rs_v, cnt_v,
               ci_last * K)

    pltpu.sync_copy(sim_v, sim_hbm.at[pl.ds(base, EPW)])
    pltpu.sync_copy(rs_v, rs_hbm.at[wid])
    pltpu.sync_copy(cnt_v, cnt_hbm.at[wid])


@functools.partial(
    pl.kernel,
    out_type=jax.ShapeDtypeStruct((E,), jnp.float32),
    mesh=_MESH,
    compiler_params=pltpu.CompilerParams(needs_layout_passes=False),
    scratch_types=[
        pltpu.VMEM((EPW,), jnp.float32),
        pltpu.VMEM((EPW,), jnp.int32),
        pltpu.VMEM((NP,), jnp.float32),
    ],
)
def _att(sim_hbm, row_hbm, invd_hbm, we_hbm, sim_v, row_v, invd_v):
    cid = lax.axis_index("c")
    sid = lax.axis_index("s")
    wid = sid * NC + cid
    base = wid * EPW
    pltpu.sync_copy(sim_hbm.at[pl.ds(base, EPW)], sim_v)
    pltpu.sync_copy(row_hbm.at[pl.ds(base, EPW)], row_v)
    pltpu.sync_copy(invd_hbm, invd_v)

    @pl.loop(0, EPW // L)
    def _we(g):
        s16 = sim_v[pl.ds(g * L, L)]
        r16 = row_v[pl.ds(g * L, L)]
        iv16 = plsc.load_gather(invd_v, [r16])
        sim_v[pl.ds(g * L, L)] = jnp.where(
            s16 > 0, jnp.exp(s16 * iv16), 0.0)

    pltpu.sync_copy(sim_v, we_hbm.at[pl.ds(base, EPW)])


@functools.partial(
    pl.kernel,
    out_type=jax.ShapeDtypeStruct((NC, NP, D), jnp.float32),
    mesh=_MESH,
    compiler_params=pltpu.CompilerParams(needs_layout_passes=False),
    scratch_types=[
        pltpu.VMEM((EPW,), jnp.int32),
        pltpu.VMEM((EPW,), jnp.int32),
        pltpu.VMEM((K,), jnp.float32),
        pltpu.VMEM((K,), jnp.float32),
        pltpu.VMEM((K, D), jnp.float32),
        pltpu.VMEM((K, D), jnp.float32),
        pltpu.VMEM_SHARED((NP, D), jnp.float32),
        pltpu.SemaphoreType.DMA,
        pltpu.SemaphoreType.DMA,
        pltpu.SemaphoreType.DMA,
        pltpu.SemaphoreType.DMA,
    ],
)
def _pass_b(sup_hbm, row_hbm, col_hbm, we_hbm, agg_hbm,
            row_v, col_v, we0, we1, rows0, rows1, agg_sh,
            semr0, semr1, semw0, semw1):
    cid = lax.axis_index("c")
    sid = lax.axis_index("s")
    wid = sid * NC + cid
    base = wid * EPW
    zero16 = jnp.zeros((L,), jnp.float32)
    rows = (rows0, rows1)
    webuf = (we0, we1)
    semr = (semr0, semr1)
    semw = (semw0, semw1)

    pltpu.sync_copy(row_hbm.at[pl.ds(base, EPW)], row_v)
    pltpu.sync_copy(col_hbm.at[pl.ds(base, EPW)], col_v)

    def _issue(ci, b):
        off = pl.multiple_of(ci * K, K)
        pltpu.async_copy(sup_hbm.at[col_v.at[pl.ds(off, K)]], rows[b], semr[b])
        pltpu.async_copy(we_hbm.at[pl.ds(base + off, K)], webuf[b], semw[b])

    def _wait(ci, b):
        off = pl.multiple_of(ci * K, K)
        pltpu.make_async_copy(sup_hbm.at[col_v.at[pl.ds(off, K)]],
                              rows[b], semr[b]).wait()
        pltpu.make_async_copy(we_hbm.at[pl.ds(base + off, K)],
                              webuf[b], semw[b]).wait()

    _issue(0, 0)

    @pl.loop(0, K)
    def _zrows(i):
        for j in range(D // L):
            rows1[i, pl.ds(j * L, L)] = zero16

    @pl.loop(0, RPS // K)
    def _zsh(j):
        pltpu.sync_copy(rows1, agg_sh.at[pl.ds(sid * RPS + j * K, K)])

    plsc.subcore_barrier()

    def _scale_scatter(ci, b):
        @pl.loop(0, K)
        def _scale(e):
            w = plsc.load_gather(webuf[b], [jnp.full((L,), e, jnp.int32)])
            for j in range(D // L):
                rows[b][e, pl.ds(j * L, L)] = rows[b][e, pl.ds(j * L, L)] * w

        off = pl.multiple_of(ci * K, K)
        pltpu.sync_copy(rows[b], agg_sh.at[row_v.at[pl.ds(off, K)]],
                        add=True)

    @pl.loop(0, (NCH - 1) // 2)
    def _chunk(j):
        for b in (0, 1):
            ci = j * 2 + b
            _wait(ci, b)
            _issue(ci + 1, 1 - b)
            _scale_scatter(ci, b)

    ci_last = NCH - 1
    _wait(ci_last, ci_last % 2)
    _scale_scatter(ci_last, ci_last % 2)

    plsc.subcore_barrier()

    @pl.loop(0, RPS // K)
    def _out(j):
        r0 = sid * RPS + j * K
        pltpu.sync_copy(agg_sh.at[pl.ds(r0, K)],
                        agg_hbm.at[cid, pl.ds(r0, K)])


def kernel(x, edge_index, W1, b1, W2, b2):
    row = edge_index[0]
    col = edge_index[1]
    xp = jnp.pad(x, ((0, NP - N), (0, 0)))
    b1r = b1.reshape(1, D)
    b2r = b2.reshape(1, D)

    xn, sup1 = _prep(xp, W1)
    rs1, cnt1, sim1 = _pass_a(xn, row, col)
    invd1, wd1 = _stats(rs1, cnt1)
    we1 = _att(sim1, row, invd1.reshape(NP))
    agg1 = _pass_b(sup1, row, col, we1)
    hn, sup2 = _mid(agg1[0], agg1[1], sup1, wd1, b1r, W2)
    rs2, cnt2, sim2 = _pass_a(hn, row, col)
    invd2, wd2 = _stats(rs2, cnt2)
    we2 = _att(sim2, row, invd2.reshape(NP))
    agg2 = _pass_b(sup2, row, col, we2)
    out = _fin(agg2[0], agg2[1], sup2, wd2, b2r)
    return out[:N]


# speedup vs baseline: 8.7448x; 2.2671x over previous
import functools

import jax
import jax.numpy as jnp
from jax import lax
from jax.experimental import pallas as pl
from jax.experimental.pallas import tpu as pltpu
from jax.experimental.pallas import tpu_sc as plsc

N = 10000
NP = 10240
E = 320000
D = 128
NC, NS, L = 2, 16, 16
NW = NC * NS
EPW = E // NW
K = 80
NCH = EPW // K
BLK = 128
G = NP // BLK
RPS = NP // NS

_HI = lax.Precision.HIGHEST


def _prep_body(x_ref, w_ref, xn_ref, sup_ref):
    xb = x_ref[...]
    nrm2 = jnp.sum(xb * xb, axis=1, keepdims=True)
    scale = jnp.where(nrm2 > 0, lax.rsqrt(nrm2), 1.0)
    xn_ref[...] = xb * scale
    sup_ref[...] = jnp.dot(xb, w_ref[...], precision=_HI,
                           preferred_element_type=jnp.float32)


def _prep(xp, W):
    return pl.pallas_call(
        _prep_body,
        grid=(G,),
        in_specs=[pl.BlockSpec((BLK, D), lambda i: (i, 0)),
                  pl.BlockSpec((D, D), lambda i: (0, 0))],
        out_specs=[pl.BlockSpec((BLK, D), lambda i: (i, 0)),
                   pl.BlockSpec((BLK, D), lambda i: (i, 0))],
        out_shape=[jax.ShapeDtypeStruct((NP, D), jnp.float32),
                   jax.ShapeDtypeStruct((NP, D), jnp.float32)],
    )(xp, W)


def _stats_body(rs_ref, cnt_ref, invd_ref, wd_ref):
    rs = jnp.sum(rs_ref[...], axis=0, keepdims=True)
    deg = jnp.sum(cnt_ref[...], axis=0, keepdims=True)
    invd_ref[0] = jnp.where(rs == 0, 1.0, 1.0 / rs)
    wd_ref[0] = jnp.exp(1.0 / (deg + 1.0))


def _stats(rs, cnt):
    return pl.pallas_call(
        _stats_body,
        grid=(G,),
        in_specs=[pl.BlockSpec((NW, BLK), lambda i: (0, i)),
                  pl.BlockSpec((NW, BLK), lambda i: (0, i))],
        out_specs=[pl.BlockSpec((1, 1, BLK), lambda i: (i, 0, 0)),
                   pl.BlockSpec((1, 1, BLK), lambda i: (i, 0, 0))],
        out_shape=[jax.ShapeDtypeStruct((G, 1, BLK), jnp.float32),
                   jax.ShapeDtypeStruct((G, 1, BLK), jnp.float32)],
    )(rs, cnt)


def _col(wd_row):
    eye = jnp.eye(BLK, dtype=jnp.float32)
    return lax.dot_general(eye, wd_row, (((1,), (1,)), ((), ())),
                           precision=_HI)


def _mid_body(a0_ref, a1_ref, sup_ref, wd_ref, b_ref, w2_ref,
              hn_ref, sup2_ref):
    wd_col = _col(wd_ref[0])
    h = a0_ref[...] + a1_ref[...] + wd_col * sup_ref[...] + b_ref[...]
    h = jnp.maximum(h, 0.0)
    nrm2 = jnp.sum(h * h, axis=1, keepdims=True)
    scale = jnp.where(nrm2 > 0, lax.rsqrt(nrm2), 1.0)
    hn_ref[...] = h * scale
    sup2_ref[...] = jnp.dot(h, w2_ref[...], precision=_HI,
                            preferred_element_type=jnp.float32)


def _mid(a0, a1, sup, wd, brow, W2):
    return pl.pallas_call(
        _mid_body,
        grid=(G,),
        in_specs=[pl.BlockSpec((BLK, D), lambda i: (i, 0)),
                  pl.BlockSpec((BLK, D), lambda i: (i, 0)),
                  pl.BlockSpec((BLK, D), lambda i: (i, 0)),
                  pl.BlockSpec((1, 1, BLK), lambda i: (i, 0, 0)),
                  pl.BlockSpec((1, D), lambda i: (0, 0)),
                  pl.BlockSpec((D, D), lambda i: (0, 0))],
        out_specs=[pl.BlockSpec((BLK, D), lambda i: (i, 0)),
                   pl.BlockSpec((BLK, D), lambda i: (i, 0))],
        out_shape=[jax.ShapeDtypeStruct((NP, D), jnp.float32),
                   jax.ShapeDtypeStruct((NP, D), jnp.float32)],
    )(a0, a1, sup, wd, brow, W2)


def _fin_body(a0_ref, a1_ref, sup_ref, wd_ref, b_ref, out_ref):
    wd_col = _col(wd_ref[0])
    out_ref[...] = a0_ref[...] + a1_ref[...] + wd_col * sup_ref[...] + b_ref[...]


def _fin(a0, a1, sup, wd, brow):
    return pl.pallas_call(
        _fin_body,
        grid=(G,),
        in_specs=[pl.BlockSpec((BLK, D), lambda i: (i, 0)),
                  pl.BlockSpec((BLK, D), lambda i: (i, 0)),
                  pl.BlockSpec((BLK, D), lambda i: (i, 0)),
                  pl.BlockSpec((1, 1, BLK), lambda i: (i, 0, 0)),
                  pl.BlockSpec((1, D), lambda i: (0, 0))],
        out_specs=pl.BlockSpec((BLK, D), lambda i: (i, 0)),
        out_shape=jax.ShapeDtypeStruct((NP, D), jnp.float32),
    )(a0, a1, sup, wd, brow)


_MESH = plsc.VectorSubcoreMesh(core_axis_name="c", subcore_axis_name="s")


def _dot_chunk(xr_v, xc_v, sim_v, row_v, rs_v, cnt_v, kbase):
    lanes = lax.iota(jnp.int32, L)

    @pl.loop(0, K // L)
    def _grp(g):
        gbase = g * L
        accv = jnp.zeros((L,), jnp.float32)
        for e in range(L):
            ge = gbase + e
            m = []
            for j in range(D // L):
                a = xr_v[ge, pl.ds(j * L, L)]
                b = xc_v[ge, pl.ds(j * L, L)]
                m.append(a * b)
            t0 = (m[0] + m[1]) + (m[2] + m[3])
            t1 = (m[4] + m[5]) + (m[6] + m[7])
            s = jnp.sum(t0 + t1)
            accv = jnp.where(lanes == e, jnp.full((L,), s, jnp.float32),
                             accv)
        sim16 = jnp.where(accv < 0.1, 0.0, accv)
        sim_v[pl.ds(kbase + gbase, L)] = sim16
        r16 = row_v[pl.ds(kbase + gbase, L)]
        plsc.addupdate_scatter(rs_v, [r16], sim16)
        plsc.addupdate_scatter(cnt_v, [r16],
                               (sim16 > 0).astype(jnp.float32))


@functools.partial(
    pl.kernel,
    out_type=(
        jax.ShapeDtypeStruct((NW, NP), jnp.float32),
        jax.ShapeDtypeStruct((NW, NP), jnp.float32),
        jax.ShapeDtypeStruct((E,), jnp.float32),
    ),
    mesh=_MESH,
    compiler_params=pltpu.CompilerParams(needs_layout_passes=False),
    scratch_types=[
        pltpu.VMEM((EPW,), jnp.int32),
        pltpu.VMEM((EPW,), jnp.int32),
        pltpu.VMEM((EPW,), jnp.float32),
        pltpu.VMEM((K, D), jnp.float32),
        pltpu.VMEM((K, D), jnp.float32),
        pltpu.VMEM((K, D), jnp.float32),
        pltpu.VMEM((K, D), jnp.float32),
        pltpu.VMEM((NP,), jnp.float32),
        pltpu.VMEM((NP,), jnp.float32),
        pltpu.SemaphoreType.DMA,
        pltpu.SemaphoreType.DMA,
        pltpu.SemaphoreType.DMA,
        pltpu.SemaphoreType.DMA,
    ],
)
def _pass_a(xn_hbm, row_hbm, col_hbm, rs_hbm, cnt_hbm, sim_hbm,
            row_v, col_v, sim_v, xr0, xc0, xr1, xc1,
            rs_v, cnt_v, semr0, semc0, semr1, semc1):
    cid = lax.axis_index("c")
    sid = lax.axis_index("s")
    wid = sid * NC + cid
    base = wid * EPW
    zero16 = jnp.zeros((L,), jnp.float32)
    xr = (xr0, xr1)
    xc = (xc0, xc1)
    semr = (semr0, semr1)
    semc = (semc0, semc1)

    pltpu.sync_copy(row_hbm.at[pl.ds(base, EPW)], row_v)
    pltpu.sync_copy(col_hbm.at[pl.ds(base, EPW)], col_v)

    @pl.loop(0, NP // L)
    def _zero(i):
        rs_v[pl.ds(i * L, L)] = zero16
        cnt_v[pl.ds(i * L, L)] = zero16

    def _issue(ci, b):
        off = pl.multiple_of(ci * K, K)
        pltpu.async_copy(xn_hbm.at[row_v.at[pl.ds(off, K)]], xr[b], semr[b])
        pltpu.async_copy(xn_hbm.at[col_v.at[pl.ds(off, K)]], xc[b], semc[b])

    def _wait(ci, b):
        off = pl.multiple_of(ci * K, K)
        pltpu.make_async_copy(xn_hbm.at[row_v.at[pl.ds(off, K)]],
                              xr[b], semr[b]).wait()
        pltpu.make_async_copy(xn_hbm.at[col_v.at[pl.ds(off, K)]],
                              xc[b], semc[b]).wait()

    _issue(0, 0)

    @pl.loop(0, (NCH - 1) // 2)
    def _chunk(j):
        for b in (0, 1):
            ci = j * 2 + b
            _wait(ci, b)
            _issue(ci + 1, 1 - b)
            _dot_chunk(xr[b], xc[b], sim_v, row_v, rs_v, cnt_v, ci * K)

    ci_last = NCH - 1
    _wait(ci_last, ci_last % 2)
    _dot_chunk(xr[ci_last % 2], xc[ci_last % 2], sim_v, row_v, rs_v, cnt_v,
               ci_last * K)

    pltpu.sync_copy(sim_v, sim_hbm.at[pl.ds(base, EPW)])
    pltpu.sync_copy(rs_v, rs_hbm.at[wid])
    pltpu.sync_copy(cnt_v, cnt_hbm.at[wid])


@functools.partial(
    pl.kernel,
    out_type=jax.ShapeDtypeStruct((E,), jnp.float32),
    mesh=_MESH,
    compiler_params=pltpu.CompilerParams(needs_layout_passes=False),
    scratch_types=[
        pltpu.VMEM((EPW,), jnp.float32),
        pltpu.VMEM((EPW,), jnp.int32),
        pltpu.VMEM((NP,), jnp.float32),
    ],
)
def _att(sim_hbm, row_hbm, invd_hbm, we_hbm, sim_v, row_v, invd_v):
    cid = lax.axis_index("c")
    sid = lax.axis_index("s")
    wid = sid * NC + cid
    base = wid * EPW
    pltpu.sync_copy(sim_hbm.at[pl.ds(base, EPW)], sim_v)
    pltpu.sync_copy(row_hbm.at[pl.ds(base, EPW)], row_v)
    pltpu.sync_copy(invd_hbm, invd_v)

    @pl.loop(0, EPW // L)
    def _we(g):
        s16 = sim_v[pl.ds(g * L, L)]
        r16 = row_v[pl.ds(g * L, L)]
        iv16 = plsc.load_gather(invd_v, [r16])
        sim_v[pl.ds(g * L, L)] = jnp.where(
            s16 > 0, jnp.exp(s16 * iv16), 0.0)

    pltpu.sync_copy(sim_v, we_hbm.at[pl.ds(base, EPW)])


@functools.partial(
    pl.kernel,
    out_type=jax.ShapeDtypeStruct((NC, NP, D), jnp.float32),
    mesh=_MESH,
    compiler_params=pltpu.CompilerParams(needs_layout_passes=False),
    scratch_types=[
        pltpu.VMEM((EPW,), jnp.int32),
        pltpu.VMEM((EPW,), jnp.int32),
        pltpu.VMEM((K,), jnp.float32),
        pltpu.VMEM((K,), jnp.float32),
        pltpu.VMEM((K, D), jnp.float32),
        pltpu.VMEM((K, D), jnp.float32),
        pltpu.VMEM_SHARED((NP, D), jnp.float32),
        pltpu.SemaphoreType.DMA,
        pltpu.SemaphoreType.DMA,
        pltpu.SemaphoreType.DMA,
        pltpu.SemaphoreType.DMA,
    ],
)
def _pass_b(sup_hbm, row_hbm, col_hbm, we_hbm, agg_hbm,
            row_v, col_v, we0, we1, rows0, rows1, agg_sh,
            semr0, semr1, semw0, semw1):
    cid = lax.axis_index("c")
    sid = lax.axis_index("s")
    wid = sid * NC + cid
    base = wid * EPW
    zero16 = jnp.zeros((L,), jnp.float32)
    rows = (rows0, rows1)
    webuf = (we0, we1)
    semr = (semr0, semr1)
    semw = (semw0, semw1)

    pltpu.sync_copy(row_hbm.at[pl.ds(base, EPW)], row_v)
    pltpu.sync_copy(col_hbm.at[pl.ds(base, EPW)], col_v)

    def _issue(ci, b):
        off = pl.multiple_of(ci * K, K)
        pltpu.async_copy(sup_hbm.at[col_v.at[pl.ds(off, K)]], rows[b], semr[b])
        pltpu.async_copy(we_hbm.at[pl.ds(base + off, K)], webuf[b], semw[b])

    def _wait(ci, b):
        off = pl.multiple_of(ci * K, K)
        pltpu.make_async_copy(sup_hbm.at[col_v.at[pl.ds(off, K)]],
                              rows[b], semr[b]).wait()
        pltpu.make_async_copy(we_hbm.at[pl.ds(base + off, K)],
                              webuf[b], semw[b]).wait()

    _issue(0, 0)

    @pl.loop(0, K)
    def _zrows(i):
        for j in range(D // L):
            rows1[i, pl.ds(j * L, L)] = zero16

    @pl.loop(0, RPS // K)
    def _zsh(j):
        pltpu.sync_copy(rows1, agg_sh.at[pl.ds(sid * RPS + j * K, K)])

    plsc.subcore_barrier()

    def _scale_scatter(ci, b):
        @pl.loop(0, K)
        def _scale(e):
            w = plsc.load_gather(webuf[b], [jnp.full((L,), e, jnp.int32)])
            for j in range(D // L):
                rows[b][e, pl.ds(j * L, L)] = rows[b][e, pl.ds(j * L, L)] * w

        off = pl.multiple_of(ci * K, K)
        pltpu.sync_copy(rows[b], agg_sh.at[row_v.at[pl.ds(off, K)]],
                        add=True)

    @pl.loop(0, (NCH - 1) // 2)
    def _chunk(j):
        for b in (0, 1):
            ci = j * 2 + b
            _wait(ci, b)
            _issue(ci + 1, 1 - b)
            _scale_scatter(ci, b)

    ci_last = NCH - 1
    _wait(ci_last, ci_last % 2)
    _scale_scatter(ci_last, ci_last % 2)

    plsc.subcore_barrier()

    @pl.loop(0, RPS // K)
    def _out(j):
        r0 = sid * RPS + j * K
        pltpu.sync_copy(agg_sh.at[pl.ds(r0, K)],
                        agg_hbm.at[cid, pl.ds(r0, K)])


def kernel(x, edge_index, W1, b1, W2, b2):
    row = edge_index[0]
    col = edge_index[1]
    xp = jnp.pad(x, ((0, NP - N), (0, 0)))
    b1r = b1.reshape(1, D)
    b2r = b2.reshape(1, D)

    xn, sup1 = _prep(xp, W1)
    rs1, cnt1, sim1 = _pass_a(xn, row, col)
    invd1, wd1 = _stats(rs1, cnt1)
    we1 = _att(sim1, row, invd1.reshape(NP))
    agg1 = _pass_b(sup1, row, col, we1)
    hn, sup2 = _mid(agg1[0], agg1[1], sup1, wd1, b1r, W2)
    rs2, cnt2, sim2 = _pass_a(hn, row, col)
    invd2, wd2 = _stats(rs2, cnt2)
    we2 = _att(sim2, row, invd2.reshape(NP))
    agg2 = _pass_b(sup2, row, col, we2)
    out = _fin(agg2[0], agg2[1], sup2, wd2, b2r)
    return out[:N]


# speedup vs baseline: 10.5239x; 1.2035x over previous
import functools

import jax
import jax.numpy as jnp
from jax import lax
from jax.experimental import pallas as pl
from jax.experimental.pallas import tpu as pltpu
from jax.experimental.pallas import tpu_sc as plsc

N = 10000
NP = 10240
E = 320000
D = 128
NC, NS, L = 2, 16, 16
NW = NC * NS
EPW = E // NW
K = 80
NCH = EPW // K
BLK = 128
G = NP // BLK
RPS = NP // NS

_HI = lax.Precision.HIGHEST


def _prep_body(x_ref, w_ref, xn_ref, sup_ref):
    xb = x_ref[...]
    nrm2 = jnp.sum(xb * xb, axis=1, keepdims=True)
    scale = jnp.where(nrm2 > 0, lax.rsqrt(nrm2), 1.0)
    xn_ref[...] = xb * scale
    sup_ref[...] = jnp.dot(xb, w_ref[...], precision=_HI,
                           preferred_element_type=jnp.float32)


def _prep(xp, W):
    return pl.pallas_call(
        _prep_body,
        grid=(G,),
        in_specs=[pl.BlockSpec((BLK, D), lambda i: (i, 0)),
                  pl.BlockSpec((D, D), lambda i: (0, 0))],
        out_specs=[pl.BlockSpec((BLK, D), lambda i: (i, 0)),
                   pl.BlockSpec((BLK, D), lambda i: (i, 0))],
        out_shape=[jax.ShapeDtypeStruct((NP, D), jnp.float32),
                   jax.ShapeDtypeStruct((NP, D), jnp.float32)],
    )(xp, W)


def _stats_body(rs_ref, cnt_ref, invd_ref, wd_ref):
    rs = jnp.sum(rs_ref[...], axis=0, keepdims=True)
    deg = jnp.sum(cnt_ref[...], axis=0, keepdims=True)
    invd_ref[0] = jnp.where(rs == 0, 1.0, 1.0 / rs)
    wd_ref[0] = jnp.exp(1.0 / (deg + 1.0))


def _stats(rs, cnt):
    return pl.pallas_call(
        _stats_body,
        grid=(G,),
        in_specs=[pl.BlockSpec((NW, BLK), lambda i: (0, i)),
                  pl.BlockSpec((NW, BLK), lambda i: (0, i))],
        out_specs=[pl.BlockSpec((1, 1, BLK), lambda i: (i, 0, 0)),
                   pl.BlockSpec((1, 1, BLK), lambda i: (i, 0, 0))],
        out_shape=[jax.ShapeDtypeStruct((G, 1, BLK), jnp.float32),
                   jax.ShapeDtypeStruct((G, 1, BLK), jnp.float32)],
    )(rs, cnt)


def _col(wd_row):
    eye = jnp.eye(BLK, dtype=jnp.float32)
    return lax.dot_general(eye, wd_row, (((1,), (1,)), ((), ())),
                           precision=_HI)


def _mid_body(a0_ref, a1_ref, sup_ref, wd_ref, b_ref, w2_ref,
              hn_ref, sup2_ref):
    wd_col = _col(wd_ref[0])
    h = a0_ref[...] + a1_ref[...] + wd_col * sup_ref[...] + b_ref[...]
    h = jnp.maximum(h, 0.0)
    nrm2 = jnp.sum(h * h, axis=1, keepdims=True)
    scale = jnp.where(nrm2 > 0, lax.rsqrt(nrm2), 1.0)
    hn_ref[...] = h * scale
    sup2_ref[...] = jnp.dot(h, w2_ref[...], precision=_HI,
                            preferred_element_type=jnp.float32)


def _mid(a0, a1, sup, wd, brow, W2):
    return pl.pallas_call(
        _mid_body,
        grid=(G,),
        in_specs=[pl.BlockSpec((BLK, D), lambda i: (i, 0)),
                  pl.BlockSpec((BLK, D), lambda i: (i, 0)),
                  pl.BlockSpec((BLK, D), lambda i: (i, 0)),
                  pl.BlockSpec((1, 1, BLK), lambda i: (i, 0, 0)),
                  pl.BlockSpec((1, D), lambda i: (0, 0)),
                  pl.BlockSpec((D, D), lambda i: (0, 0))],
        out_specs=[pl.BlockSpec((BLK, D), lambda i: (i, 0)),
                   pl.BlockSpec((BLK, D), lambda i: (i, 0))],
        out_shape=[jax.ShapeDtypeStruct((NP, D), jnp.float32),
                   jax.ShapeDtypeStruct((NP, D), jnp.float32)],
    )(a0, a1, sup, wd, brow, W2)


def _fin_body(a0_ref, a1_ref, sup_ref, wd_ref, b_ref, out_ref):
    wd_col = _col(wd_ref[0])
    out_ref[...] = a0_ref[...] + a1_ref[...] + wd_col * sup_ref[...] + b_ref[...]


def _fin(a0, a1, sup, wd, brow):
    return pl.pallas_call(
        _fin_body,
        grid=(G,),
        in_specs=[pl.BlockSpec((BLK, D), lambda i: (i, 0)),
                  pl.BlockSpec((BLK, D), lambda i: (i, 0)),
                  pl.BlockSpec((BLK, D), lambda i: (i, 0)),
                  pl.BlockSpec((1, 1, BLK), lambda i: (i, 0, 0)),
                  pl.BlockSpec((1, D), lambda i: (0, 0))],
        out_specs=pl.BlockSpec((BLK, D), lambda i: (i, 0)),
        out_shape=jax.ShapeDtypeStruct((NP, D), jnp.float32),
    )(a0, a1, sup, wd, brow)


_MESH = plsc.VectorSubcoreMesh(core_axis_name="c", subcore_axis_name="s")


def _dot_chunk(xr_v, xc_v, sim_v, row_v, rs_v, cnt_v, stage_v, kbase):
    base17 = lax.iota(jnp.int32, L) * (L + 1)

    @pl.loop(0, K // L)
    def _grp(g):
        gbase = g * L
        for e in range(L):
            ge = gbase + e
            m = []
            for j in range(D // L):
                a = xr_v[ge, pl.ds(j * L, L)]
                b = xc_v[ge, pl.ds(j * L, L)]
                m.append(a * b)
            t0 = (m[0] + m[1]) + (m[2] + m[3])
            t1 = (m[4] + m[5]) + (m[6] + m[7])
            stage_v[pl.ds(e * (L + 1), L)] = t0 + t1
        cols = [plsc.load_gather(stage_v, [base17 + d]) for d in range(L)]
        while len(cols) > 1:
            cols = [cols[i] + cols[i + 1] for i in range(0, len(cols), 2)]
        accv = cols[0]
        sim16 = jnp.where(accv < 0.1, 0.0, accv)
        sim_v[pl.ds(kbase + gbase, L)] = sim16
        r16 = row_v[pl.ds(kbase + gbase, L)]
        plsc.addupdate_scatter(rs_v, [r16], sim16)
        plsc.addupdate_scatter(cnt_v, [r16],
                               (sim16 > 0).astype(jnp.float32))


@functools.partial(
    pl.kernel,
    out_type=(
        jax.ShapeDtypeStruct((NW, NP), jnp.float32),
        jax.ShapeDtypeStruct((NW, NP), jnp.float32),
        jax.ShapeDtypeStruct((E,), jnp.float32),
    ),
    mesh=_MESH,
    compiler_params=pltpu.CompilerParams(needs_layout_passes=False),
    scratch_types=[
        pltpu.VMEM((EPW,), jnp.int32),
        pltpu.VMEM((EPW,), jnp.int32),
        pltpu.VMEM((EPW,), jnp.float32),
        pltpu.VMEM((K, D), jnp.float32),
        pltpu.VMEM((K, D), jnp.float32),
        pltpu.VMEM((K, D), jnp.float32),
        pltpu.VMEM((K, D), jnp.float32),
        pltpu.VMEM((NP,), jnp.float32),
        pltpu.VMEM((NP,), jnp.float32),
        pltpu.VMEM((L * (L + 1),), jnp.float32),
        pltpu.SemaphoreType.DMA,
        pltpu.SemaphoreType.DMA,
        pltpu.SemaphoreType.DMA,
        pltpu.SemaphoreType.DMA,
    ],
)
def _pass_a(xn_hbm, row_hbm, col_hbm, rs_hbm, cnt_hbm, sim_hbm,
            row_v, col_v, sim_v, xr0, xc0, xr1, xc1,
            rs_v, cnt_v, stage_v, semr0, semc0, semr1, semc1):
    cid = lax.axis_index("c")
    sid = lax.axis_index("s")
    wid = sid * NC + cid
    base = wid * EPW
    zero16 = jnp.zeros((L,), jnp.float32)
    xr = (xr0, xr1)
    xc = (xc0, xc1)
    semr = (semr0, semr1)
    semc = (semc0, semc1)

    pltpu.sync_copy(row_hbm.at[pl.ds(base, EPW)], row_v)
    pltpu.sync_copy(col_hbm.at[pl.ds(base, EPW)], col_v)

    @pl.loop(0, NP // L)
    def _zero(i):
        rs_v[pl.ds(i * L, L)] = zero16
        cnt_v[pl.ds(i * L, L)] = zero16

    def _issue(ci, b):
        off = pl.multiple_of(ci * K, K)
        pltpu.async_copy(xn_hbm.at[row_v.at[pl.ds(off, K)]], xr[b], semr[b])
        pltpu.async_copy(xn_hbm.at[col_v.at[pl.ds(off, K)]], xc[b], semc[b])

    def _wait(ci, b):
        off = pl.multiple_of(ci * K, K)
        pltpu.make_async_copy(xn_hbm.at[row_v.at[pl.ds(off, K)]],
                              xr[b], semr[b]).wait()
        pltpu.make_async_copy(xn_hbm.at[col_v.at[pl.ds(off, K)]],
                              xc[b], semc[b]).wait()

    _issue(0, 0)

    @pl.loop(0, (NCH - 1) // 2)
    def _chunk(j):
        for b in (0, 1):
            ci = j * 2 + b
            _wait(ci, b)
            _issue(ci + 1, 1 - b)
            _dot_chunk(xr[b], xc[b], sim_v, row_v, rs_v, cnt_v, stage_v, ci * K)

    ci_last = NCH - 1
    _wait(ci_last, ci_last % 2)
    _dot_chunk(xr[ci_last % 2], xc[ci_last % 2], sim_v, row_v, rs_v, cnt_v,
               stage_v, ci_last * K)

    pltpu.sync_copy(sim_v, sim_hbm.at[pl.ds(base, EPW)])
    pltpu.sync_copy(rs_v, rs_hbm.at[wid])
    pltpu.sync_copy(cnt_v, cnt_hbm.at[wid])


@functools.partial(
    pl.kernel,
    out_type=jax.ShapeDtypeStruct((E,), jnp.float32),
    mesh=_MESH,
    compiler_params=pltpu.CompilerParams(needs_layout_passes=False),
    scratch_types=[
        pltpu.VMEM((EPW,), jnp.float32),
        pltpu.VMEM((EPW,), jnp.int32),
        pltpu.VMEM((NP,), jnp.float32),
    ],
)
def _att(sim_hbm, row_hbm, invd_hbm, we_hbm, sim_v, row_v, invd_v):
    cid = lax.axis_index("c")
    sid = lax.axis_index("s")
    wid = sid * NC + cid
    base = wid * EPW
    pltpu.sync_copy(sim_hbm.at[pl.ds(base, EPW)], sim_v)
    pltpu.sync_copy(row_hbm.at[pl.ds(base, EPW)], row_v)
    pltpu.sync_copy(invd_hbm, invd_v)

    @pl.loop(0, EPW // L)
    def _we(g):
        s16 = sim_v[pl.ds(g * L, L)]
        r16 = row_v[pl.ds(g * L, L)]
        iv16 = plsc.load_gather(invd_v, [r16])
        sim_v[pl.ds(g * L, L)] = jnp.where(
            s16 > 0, jnp.exp(s16 * iv16), 0.0)

    pltpu.sync_copy(sim_v, we_hbm.at[pl.ds(base, EPW)])


@functools.partial(
    pl.kernel,
    out_type=jax.ShapeDtypeStruct((NC, NP, D), jnp.float32),
    mesh=_MESH,
    compiler_params=pltpu.CompilerParams(needs_layout_passes=False),
    scratch_types=[
        pltpu.VMEM((EPW,), jnp.int32),
        pltpu.VMEM((EPW,), jnp.int32),
        pltpu.VMEM((K,), jnp.float32),
        pltpu.VMEM((K,), jnp.float32),
        pltpu.VMEM((K, D), jnp.float32),
        pltpu.VMEM((K, D), jnp.float32),
        pltpu.VMEM_SHARED((NP, D), jnp.float32),
        pltpu.SemaphoreType.DMA,
        pltpu.SemaphoreType.DMA,
        pltpu.SemaphoreType.DMA,
        pltpu.SemaphoreType.DMA,
    ],
)
def _pass_b(sup_hbm, row_hbm, col_hbm, we_hbm, agg_hbm,
            row_v, col_v, we0, we1, rows0, rows1, agg_sh,
            semr0, semr1, semw0, semw1):
    cid = lax.axis_index("c")
    sid = lax.axis_index("s")
    wid = sid * NC + cid
    base = wid * EPW
    zero16 = jnp.zeros((L,), jnp.float32)
    rows = (rows0, rows1)
    webuf = (we0, we1)
    semr = (semr0, semr1)
    semw = (semw0, semw1)

    pltpu.sync_copy(row_hbm.at[pl.ds(base, EPW)], row_v)
    pltpu.sync_copy(col_hbm.at[pl.ds(base, EPW)], col_v)

    def _issue(ci, b):
        off = pl.multiple_of(ci * K, K)
        pltpu.async_copy(sup_hbm.at[col_v.at[pl.ds(off, K)]], rows[b], semr[b])
        pltpu.async_copy(we_hbm.at[pl.ds(base + off, K)], webuf[b], semw[b])

    def _wait(ci, b):
        off = pl.multiple_of(ci * K, K)
        pltpu.make_async_copy(sup_hbm.at[col_v.at[pl.ds(off, K)]],
                              rows[b], semr[b]).wait()
        pltpu.make_async_copy(we_hbm.at[pl.ds(base + off, K)],
                              webuf[b], semw[b]).wait()

    _issue(0, 0)

    @pl.loop(0, K)
    def _zrows(i):
        for j in range(D // L):
            rows1[i, pl.ds(j * L, L)] = zero16

    @pl.loop(0, RPS // K)
    def _zsh(j):
        pltpu.sync_copy(rows1, agg_sh.at[pl.ds(sid * RPS + j * K, K)])

    plsc.subcore_barrier()

    def _scale_scatter(ci, b):
        @pl.loop(0, K, unroll=8)
        def _scale(e):
            w = plsc.load_gather(webuf[b], [jnp.full((L,), e, jnp.int32)])
            for j in range(D // L):
                rows[b][e, pl.ds(j * L, L)] = rows[b][e, pl.ds(j * L, L)] * w

        off = pl.multiple_of(ci * K, K)
        pltpu.sync_copy(rows[b], agg_sh.at[row_v.at[pl.ds(off, K)]],
                        add=True)

    @pl.loop(0, (NCH - 1) // 2)
    def _chunk(j):
        for b in (0, 1):
            ci = j * 2 + b
            _wait(ci, b)
            _issue(ci + 1, 1 - b)
            _scale_scatter(ci, b)

    ci_last = NCH - 1
    _wait(ci_last, ci_last % 2)
    _scale_scatter(ci_last, ci_last % 2)

    plsc.subcore_barrier()

    @pl.loop(0, RPS // K)
    def _out(j):
        r0 = sid * RPS + j * K
        pltpu.sync_copy(agg_sh.at[pl.ds(r0, K)],
                        agg_hbm.at[cid, pl.ds(r0, K)])


def kernel(x, edge_index, W1, b1, W2, b2):
    row = edge_index[0]
    col = edge_index[1]
    xp = jnp.pad(x, ((0, NP - N), (0, 0)))
    b1r = b1.reshape(1, D)
    b2r = b2.reshape(1, D)

    xn, sup1 = _prep(xp, W1)
    rs1, cnt1, sim1 = _pass_a(xn, row, col)
    invd1, wd1 = _stats(rs1, cnt1)
    we1 = _att(sim1, row, invd1.reshape(NP))
    agg1 = _pass_b(sup1, row, col, we1)
    hn, sup2 = _mid(agg1[0], agg1[1], sup1, wd1, b1r, W2)
    rs2, cnt2, sim2 = _pass_a(hn, row, col)
    invd2, wd2 = _stats(rs2, cnt2)
    we2 = _att(sim2, row, invd2.reshape(NP))
    agg2 = _pass_b(sup2, row, col, we2)
    out = _fin(agg2[0], agg2[1], sup2, wd2, b2r)
    return out[:N]


# speedup vs baseline: 10.6517x; 1.0121x over previous
import functools

import jax
import jax.numpy as jnp
from jax import lax
from jax.experimental import pallas as pl
from jax.experimental.pallas import tpu as pltpu
from jax.experimental.pallas import tpu_sc as plsc

N = 10000
NP = 10240
E = 320000
D = 128
NC, NS, L = 2, 16, 16
NW = NC * NS
EPW = E // NW
K = 80
NCH = EPW // K
BLK = 128
G = NP // BLK
RPS = NP // NS

_HI = lax.Precision.HIGHEST


def _prep_body(x_ref, w_ref, xn_ref, sup_ref):
    xb = x_ref[...]
    nrm2 = jnp.sum(xb * xb, axis=1, keepdims=True)
    scale = jnp.where(nrm2 > 0, lax.rsqrt(nrm2), 1.0)
    xn_ref[...] = xb * scale
    sup_ref[...] = jnp.dot(xb, w_ref[...], precision=_HI,
                           preferred_element_type=jnp.float32)


def _prep(xp, W):
    return pl.pallas_call(
        _prep_body,
        grid=(G,),
        in_specs=[pl.BlockSpec((BLK, D), lambda i: (i, 0)),
                  pl.BlockSpec((D, D), lambda i: (0, 0))],
        out_specs=[pl.BlockSpec((BLK, D), lambda i: (i, 0)),
                   pl.BlockSpec((BLK, D), lambda i: (i, 0))],
        out_shape=[jax.ShapeDtypeStruct((NP, D), jnp.float32),
                   jax.ShapeDtypeStruct((NP, D), jnp.float32)],
    )(xp, W)


def _stats_body(rs_ref, cnt_ref, invd_ref, wd_ref):
    rs = jnp.sum(rs_ref[...], axis=0, keepdims=True)
    deg = jnp.sum(cnt_ref[...], axis=0, keepdims=True)
    invd_ref[0] = jnp.where(rs == 0, 1.0, 1.0 / rs)
    wd_ref[0] = jnp.exp(1.0 / (deg + 1.0))


def _stats(rs, cnt):
    return pl.pallas_call(
        _stats_body,
        grid=(G,),
        in_specs=[pl.BlockSpec((NW, BLK), lambda i: (0, i)),
                  pl.BlockSpec((NW, BLK), lambda i: (0, i))],
        out_specs=[pl.BlockSpec((1, 1, BLK), lambda i: (i, 0, 0)),
                   pl.BlockSpec((1, 1, BLK), lambda i: (i, 0, 0))],
        out_shape=[jax.ShapeDtypeStruct((G, 1, BLK), jnp.float32),
                   jax.ShapeDtypeStruct((G, 1, BLK), jnp.float32)],
    )(rs, cnt)


def _col(wd_row):
    eye = jnp.eye(BLK, dtype=jnp.float32)
    return lax.dot_general(eye, wd_row, (((1,), (1,)), ((), ())),
                           precision=_HI)


def _mid_body(a0_ref, a1_ref, sup_ref, wd_ref, b_ref, w2_ref,
              hn_ref, sup2_ref):
    wd_col = _col(wd_ref[0])
    h = a0_ref[...] + a1_ref[...] + wd_col * sup_ref[...] + b_ref[...]
    h = jnp.maximum(h, 0.0)
    nrm2 = jnp.sum(h * h, axis=1, keepdims=True)
    scale = jnp.where(nrm2 > 0, lax.rsqrt(nrm2), 1.0)
    hn_ref[...] = h * scale
    sup2_ref[...] = jnp.dot(h, w2_ref[...], precision=_HI,
                            preferred_element_type=jnp.float32)


def _mid(a0, a1, sup, wd, brow, W2):
    return pl.pallas_call(
        _mid_body,
        grid=(G,),
        in_specs=[pl.BlockSpec((BLK, D), lambda i: (i, 0)),
                  pl.BlockSpec((BLK, D), lambda i: (i, 0)),
                  pl.BlockSpec((BLK, D), lambda i: (i, 0)),
                  pl.BlockSpec((1, 1, BLK), lambda i: (i, 0, 0)),
                  pl.BlockSpec((1, D), lambda i: (0, 0)),
                  pl.BlockSpec((D, D), lambda i: (0, 0))],
        out_specs=[pl.BlockSpec((BLK, D), lambda i: (i, 0)),
                   pl.BlockSpec((BLK, D), lambda i: (i, 0))],
        out_shape=[jax.ShapeDtypeStruct((NP, D), jnp.float32),
                   jax.ShapeDtypeStruct((NP, D), jnp.float32)],
    )(a0, a1, sup, wd, brow, W2)


def _fin_body(a0_ref, a1_ref, sup_ref, wd_ref, b_ref, out_ref):
    wd_col = _col(wd_ref[0])
    out_ref[...] = a0_ref[...] + a1_ref[...] + wd_col * sup_ref[...] + b_ref[...]


def _fin(a0, a1, sup, wd, brow):
    return pl.pallas_call(
        _fin_body,
        grid=(G,),
        in_specs=[pl.BlockSpec((BLK, D), lambda i: (i, 0)),
                  pl.BlockSpec((BLK, D), lambda i: (i, 0)),
                  pl.BlockSpec((BLK, D), lambda i: (i, 0)),
                  pl.BlockSpec((1, 1, BLK), lambda i: (i, 0, 0)),
                  pl.BlockSpec((1, D), lambda i: (0, 0))],
        out_specs=pl.BlockSpec((BLK, D), lambda i: (i, 0)),
        out_shape=jax.ShapeDtypeStruct((NP, D), jnp.float32),
    )(a0, a1, sup, wd, brow)


_MESH = plsc.VectorSubcoreMesh(core_axis_name="c", subcore_axis_name="s")


def _dot_chunk(xr_v, xc_v, sim_v, row_v, rs_v, cnt_v, stage_v, kbase):
    base17 = lax.iota(jnp.int32, L) * (L + 1)

    @pl.loop(0, K // L)
    def _grp(g):
        gbase = g * L
        for e in range(L):
            ge = gbase + e
            m = []
            for j in range(D // L):
                a = xr_v[ge, pl.ds(j * L, L)]
                b = xc_v[ge, pl.ds(j * L, L)]
                m.append(a * b)
            t0 = (m[0] + m[1]) + (m[2] + m[3])
            t1 = (m[4] + m[5]) + (m[6] + m[7])
            stage_v[pl.ds(e * (L + 1), L)] = t0 + t1
        cols = [plsc.load_gather(stage_v, [base17 + d]) for d in range(L)]
        while len(cols) > 1:
            cols = [cols[i] + cols[i + 1] for i in range(0, len(cols), 2)]
        accv = cols[0]
        sim16 = jnp.where(accv < 0.1, 0.0, accv)
        sim_v[pl.ds(kbase + gbase, L)] = sim16
        r16 = row_v[pl.ds(kbase + gbase, L)]
        plsc.addupdate_scatter(rs_v, [r16], sim16)
        plsc.addupdate_scatter(cnt_v, [r16],
                               (sim16 > 0).astype(jnp.float32))


@functools.partial(
    pl.kernel,
    out_type=(
        jax.ShapeDtypeStruct((NW, NP), jnp.float32),
        jax.ShapeDtypeStruct((NW, NP), jnp.float32),
        jax.ShapeDtypeStruct((E,), jnp.float32),
    ),
    mesh=_MESH,
    compiler_params=pltpu.CompilerParams(needs_layout_passes=False),
    scratch_types=[
        pltpu.VMEM((EPW,), jnp.int32),
        pltpu.VMEM((EPW,), jnp.int32),
        pltpu.VMEM((EPW,), jnp.float32),
        pltpu.VMEM((K, D), jnp.float32),
        pltpu.VMEM((K, D), jnp.float32),
        pltpu.VMEM((K, D), jnp.float32),
        pltpu.VMEM((K, D), jnp.float32),
        pltpu.VMEM((NP,), jnp.float32),
        pltpu.VMEM((NP,), jnp.float32),
        pltpu.VMEM((L * (L + 1),), jnp.float32),
        pltpu.SemaphoreType.DMA,
        pltpu.SemaphoreType.DMA,
        pltpu.SemaphoreType.DMA,
        pltpu.SemaphoreType.DMA,
    ],
)
def _pass_a(xn_hbm, row_hbm, col_hbm, rs_hbm, cnt_hbm, sim_hbm,
            row_v, col_v, sim_v, xr0, xc0, xr1, xc1,
            rs_v, cnt_v, stage_v, semr0, semc0, semr1, semc1):
    cid = lax.axis_index("c")
    sid = lax.axis_index("s")
    wid = sid * NC + cid
    base = wid * EPW
    zero16 = jnp.zeros((L,), jnp.float32)
    xr = (xr0, xr1)
    xc = (xc0, xc1)
    semr = (semr0, semr1)
    semc = (semc0, semc1)

    pltpu.sync_copy(row_hbm.at[pl.ds(base, EPW)], row_v)
    pltpu.sync_copy(col_hbm.at[pl.ds(base, EPW)], col_v)

    @pl.loop(0, NP // L)
    def _zero(i):
        rs_v[pl.ds(i * L, L)] = zero16
        cnt_v[pl.ds(i * L, L)] = zero16

    def _issue(ci, b):
        off = pl.multiple_of(ci * K, K)
        pltpu.async_copy(xn_hbm.at[row_v.at[pl.ds(off, K)]], xr[b], semr[b])
        pltpu.async_copy(xn_hbm.at[col_v.at[pl.ds(off, K)]], xc[b], semc[b])

    def _wait(ci, b):
        off = pl.multiple_of(ci * K, K)
        pltpu.make_async_copy(xn_hbm.at[row_v.at[pl.ds(off, K)]],
                              xr[b], semr[b]).wait()
        pltpu.make_async_copy(xn_hbm.at[col_v.at[pl.ds(off, K)]],
                              xc[b], semc[b]).wait()

    _issue(0, 0)

    @pl.loop(0, (NCH - 1) // 2)
    def _chunk(j):
        for b in (0, 1):
            ci = j * 2 + b
            _wait(ci, b)
            _issue(ci + 1, 1 - b)
            _dot_chunk(xr[b], xc[b], sim_v, row_v, rs_v, cnt_v, stage_v, ci * K)

    ci_last = NCH - 1
    _wait(ci_last, ci_last % 2)
    _dot_chunk(xr[ci_last % 2], xc[ci_last % 2], sim_v, row_v, rs_v, cnt_v,
               stage_v, ci_last * K)

    pltpu.sync_copy(sim_v, sim_hbm.at[pl.ds(base, EPW)])
    pltpu.sync_copy(rs_v, rs_hbm.at[wid])
    pltpu.sync_copy(cnt_v, cnt_hbm.at[wid])


@functools.partial(
    pl.kernel,
    out_type=jax.ShapeDtypeStruct((E,), jnp.float32),
    mesh=_MESH,
    compiler_params=pltpu.CompilerParams(needs_layout_passes=False),
    scratch_types=[
        pltpu.VMEM((EPW,), jnp.float32),
        pltpu.VMEM((EPW,), jnp.int32),
        pltpu.VMEM((NP,), jnp.float32),
    ],
)
def _att(sim_hbm, row_hbm, invd_hbm, we_hbm, sim_v, row_v, invd_v):
    cid = lax.axis_index("c")
    sid = lax.axis_index("s")
    wid = sid * NC + cid
    base = wid * EPW
    pltpu.sync_copy(sim_hbm.at[pl.ds(base, EPW)], sim_v)
    pltpu.sync_copy(row_hbm.at[pl.ds(base, EPW)], row_v)
    pltpu.sync_copy(invd_hbm, invd_v)

    @pl.loop(0, EPW // L)
    def _we(g):
        s16 = sim_v[pl.ds(g * L, L)]
        r16 = row_v[pl.ds(g * L, L)]
        iv16 = plsc.load_gather(invd_v, [r16])
        sim_v[pl.ds(g * L, L)] = jnp.where(
            s16 > 0, jnp.exp(s16 * iv16), 0.0)

    pltpu.sync_copy(sim_v, we_hbm.at[pl.ds(base, EPW)])


@functools.partial(
    pl.kernel,
    out_type=jax.ShapeDtypeStruct((NC, NP, D), jnp.float32),
    mesh=_MESH,
    compiler_params=pltpu.CompilerParams(needs_layout_passes=False),
    scratch_types=[
        pltpu.VMEM((EPW,), jnp.int32),
        pltpu.VMEM((3, K), jnp.int32),
        pltpu.VMEM((3, K), jnp.float32),
        pltpu.VMEM((K, D), jnp.float32),
        pltpu.VMEM((K, D), jnp.float32),
        pltpu.VMEM((K, D), jnp.float32),
        pltpu.VMEM_SHARED((NP, D), jnp.float32),
        pltpu.SemaphoreType.DMA,
        pltpu.SemaphoreType.DMA,
        pltpu.SemaphoreType.DMA,
        pltpu.SemaphoreType.DMA,
        pltpu.SemaphoreType.DMA,
        pltpu.SemaphoreType.DMA,
    ],
)
def _pass_b(sup_hbm, row_hbm, col_hbm, we_hbm, agg_hbm,
            col_v, rowb, web, rows0, rows1, rows2, agg_sh,
            semg0, semg1, semg2, sems0, sems1, sems2):
    cid = lax.axis_index("c")
    sid = lax.axis_index("s")
    wid = sid * NC + cid
    base = wid * EPW
    zero16 = jnp.zeros((L,), jnp.float32)
    rows = (rows0, rows1, rows2)
    semg = (semg0, semg1, semg2)
    sems = (sems0, sems1, sems2)

    pltpu.sync_copy(col_hbm.at[pl.ds(base, EPW)], col_v)

    def _issue(ci, b):
        off = pl.multiple_of(ci * K, K)
        pltpu.async_copy(sup_hbm.at[col_v.at[pl.ds(off, K)]], rows[b],
                         semg[b])
        pltpu.async_copy(row_hbm.at[pl.ds(base + off, K)], rowb.at[b],
                         semg[b])
        pltpu.async_copy(we_hbm.at[pl.ds(base + off, K)], web.at[b],
                         semg[b])

    def _wait_g(ci, b):
        off = pl.multiple_of(ci * K, K)
        pltpu.make_async_copy(sup_hbm.at[col_v.at[pl.ds(off, K)]], rows[b],
                              semg[b]).wait()
        pltpu.make_async_copy(row_hbm.at[pl.ds(base + off, K)], rowb.at[b],
                              semg[b]).wait()
        pltpu.make_async_copy(we_hbm.at[pl.ds(base + off, K)], web.at[b],
                              semg[b]).wait()

    def _scale(b):
        @pl.loop(0, K, unroll=8)
        def _s(e):
            w = plsc.load_gather(web.at[b], [jnp.full((L,), e, jnp.int32)])
            for j in range(D // L):
                rows[b][e, pl.ds(j * L, L)] = rows[b][e, pl.ds(j * L, L)] * w

    def _start_s(b):
        return pltpu.async_copy(rows[b], agg_sh.at[rowb.at[b]], sems[b])

    def _wait_s(b):
        pltpu.make_async_copy(rows[b], agg_sh.at[rowb.at[b]], sems[b]).wait()

    _issue(0, 0)

    @pl.loop(0, K)
    def _zrows(i):
        for j in range(D // L):
            rows1[i, pl.ds(j * L, L)] = zero16

    @pl.loop(0, RPS // K)
    def _zsh(j):
        pltpu.sync_copy(rows1, agg_sh.at[pl.ds(sid * RPS + j * K, K)])

    plsc.subcore_barrier()

    for ci in (0, 1, 2):
        b = ci % 3
        _wait_g(ci, b)
        if ci + 1 < NCH:
            bn = (ci + 1) % 3
            if ci >= 2:
                _wait_s(bn)
            _issue(ci + 1, bn)
        _scale(b)
        _start_s(b)

    @pl.loop(1, (NCH - 2) // 3)
    def _chunk(j):
        for t in (0, 1, 2):
            ci = j * 3 + t
            b = t
            _wait_g(ci, b)
            bn = (t + 1) % 3
            _wait_s(bn)
            _issue(ci + 1, bn)
            _scale(b)
            _start_s(b)

    for ci in (NCH - 2, NCH - 1):
        b = ci % 3
        _wait_g(ci, b)
        if ci + 1 < NCH:
            bn = (ci + 1) % 3
            _wait_s(bn)
            _issue(ci + 1, bn)
        _scale(b)
        _start_s(b)

    for ci in (NCH - 3, NCH - 2, NCH - 1):
        _wait_s(ci % 3)

    plsc.subcore_barrier()

    @pl.loop(0, RPS // K)
    def _out(j):
        r0 = sid * RPS + j * K
        pltpu.sync_copy(agg_sh.at[pl.ds(r0, K)],
                        agg_hbm.at[cid, pl.ds(r0, K)])


def kernel(x, edge_index, W1, b1, W2, b2):
    row = edge_index[0]
    col = edge_index[1]
    xp = jnp.pad(x, ((0, NP - N), (0, 0)))
    b1r = b1.reshape(1, D)
    b2r = b2.reshape(1, D)

    xn, sup1 = _prep(xp, W1)
    rs1, cnt1, sim1 = _pass_a(xn, row, col)
    invd1, wd1 = _stats(rs1, cnt1)
    we1 = _att(sim1, row, invd1.reshape(NP))
    agg1 = _pass_b(sup1, row, col, we1)
    hn, sup2 = _mid(agg1[0], agg1[1], sup1, wd1, b1r, W2)
    rs2, cnt2, sim2 = _pass_a(hn, row, col)
    invd2, wd2 = _stats(rs2, cnt2)
    we2 = _att(sim2, row, invd2.reshape(NP))
    agg2 = _pass_b(sup2, row, col, we2)
    out = _fin(agg2[0], agg2[1], sup2, wd2, b2r)
    return out[:N]
